# baseline (speedup 1.0000x reference)
import functools

import jax
import jax.numpy as jnp
from jax.experimental import pallas as pl
from jax.experimental.pallas import tpu as pltpu


def _mm_body(x_ref, w_ref, b_ref, o_ref, *, act):
    acc = jnp.dot(x_ref[...], w_ref[...], preferred_element_type=jnp.float32)
    acc = acc + b_ref[...]
    if act == "relu":
        acc = jnp.maximum(acc, 0.0)
    elif act == "elu":
        acc = jnp.where(acc > 0, acc, jnp.exp(jnp.minimum(acc, 0.0)) - 1.0)
    o_ref[...] = acc


def _mm(x, w, b=None, act="none", bm=1024):
    M, K = x.shape
    N = w.shape[1]
    if b is None:
        b = jnp.zeros((N,), jnp.float32)
    b2 = b.reshape(1, N)
    bm = min(bm, M)
    grid = (pl.cdiv(M, bm),)
    return pl.pallas_call(
        functools.partial(_mm_body, act=act),
        grid=grid,
        in_specs=[
            pl.BlockSpec((bm, K), lambda i: (i, 0)),
            pl.BlockSpec((K, N), lambda i: (0, 0)),
            pl.BlockSpec((1, N), lambda i: (0, 0)),
        ],
        out_specs=pl.BlockSpec((bm, N), lambda i: (i, 0)),
        out_shape=jax.ShapeDtypeStruct((M, N), jnp.float32),
    )(x, w, b2)


def _gat_layer(x, src, dst, W, a_src, a_dst, bias):
    N = x.shape[0]
    H, C = a_src.shape
    xw = _mm(x, W)
    xw3 = xw.reshape(N, H, C)
    alpha_src = jnp.sum(xw3 * a_src[None, :, :], axis=-1)
    alpha_dst = jnp.sum(xw3 * a_dst[None, :, :], axis=-1)

    alpha = alpha_src[src] + alpha_dst[dst]
    alpha = jnp.where(alpha >= 0, alpha, 0.2 * alpha)
    amax = jax.ops.segment_max(alpha, dst, num_segments=N)
    alpha = jnp.exp(alpha - amax[dst])
    denom = jax.ops.segment_sum(alpha, dst, num_segments=N)
    alpha = alpha / (denom[dst] + 1e-16)
    out = jax.ops.segment_sum(xw3[src] * alpha[:, :, None], dst, num_segments=N)
    return out.reshape(N, H * C) + bias


def _gcn_layer(x, src, dst, dinv, W, bias):
    N = x.shape[0]
    xw = _mm(x, W)
    norm = dinv[src] * dinv[dst]
    out = jax.ops.segment_sum(xw[src] * norm[:, None], dst, num_segments=N)
    return out + bias


def kernel(x_drug, edge_index_drug, batch_drug, x_prots, edge_index_prots,
           batch_prots, W1, a_src1, a_dst1, b1, W2, a_src2, a_dst2, b2,
           Wg1, bg1, Wp1, bp1, Wp2, bp2, Wl1, bl1, Wf1, bf1, Wf2, bf2, Wo, bo):
    ND = x_drug.shape[0]
    NP = x_prots.shape[0]
    B = 256

    loop_d = jnp.arange(ND, dtype=edge_index_drug.dtype)
    src_d = jnp.concatenate([edge_index_drug[0], loop_d])
    dst_d = jnp.concatenate([edge_index_drug[1], loop_d])
    loop_p = jnp.arange(NP, dtype=edge_index_prots.dtype)
    src_p = jnp.concatenate([edge_index_prots[0], loop_p])
    dst_p = jnp.concatenate([edge_index_prots[1], loop_p])

    x = _gat_layer(x_drug, src_d, dst_d, W1, a_src1, a_dst1, b1)
    x = jax.nn.elu(x)
    x = _gat_layer(x, src_d, dst_d, W2, a_src2, a_dst2, b2)
    x = jnp.maximum(x, 0.0)
    xg = jax.ops.segment_max(x, batch_drug, num_segments=B)
    xg = _mm(xg, Wg1, bg1, act="relu")

    deg = jax.ops.segment_sum(jnp.ones(src_p.shape[0], jnp.float32), dst_p,
                              num_segments=NP)
    dinv = jax.lax.rsqrt(jnp.maximum(deg, 1.0))
    xp = _gcn_layer(x_prots, src_p, dst_p, dinv, Wp1, bp1)
    xp = jnp.maximum(xp, 0.0)
    xp = _gcn_layer(xp, src_p, dst_p, dinv, Wp2, bp2)
    xp = jnp.maximum(xp, 0.0)
    xpg = jax.ops.segment_max(xp, batch_prots, num_segments=B)
    xpg = _mm(xpg, Wl1, bl1, act="relu")

    xc = jnp.concatenate([xg, xpg], axis=1)
    xc = _mm(xc, Wf1, bf1, act="relu")
    xc = _mm(xc, Wf2, bf2, act="relu")
    Wo_p = jnp.pad(Wo, ((0, 0), (0, 127)))
    bo_p = jnp.pad(bo, (0, 127))
    out = _mm(xc, Wo_p, bo_p)[:, :1]
    return out
